# baseline (speedup 1.0000x reference)
import functools

import jax
import jax.numpy as jnp
from jax import lax
from jax.experimental import pallas as pl
from jax.experimental.pallas import tpu as pltpu
from jax.experimental.pallas import tpu_sc as plsc


def _halo_copies(x_any, buf, sem, s, nt, *, R, W, H):
  b2 = s // nt
  t2 = lax.rem(s, nt)
  slot = lax.rem(s, 2)
  row0 = t2 * R
  top = jnp.maximum(row0 - 1, 0)
  bot = jnp.minimum(row0 + R, H - 1)
  return (
      pltpu.make_async_copy(
          x_any.at[b2, pl.ds(row0, R)],
          buf.at[slot, pl.ds(1, R), pl.ds(1, W)], sem.at[slot]),
      pltpu.make_async_copy(
          x_any.at[b2, pl.ds(top, 1)],
          buf.at[slot, pl.ds(0, 1), pl.ds(1, W)], sem.at[slot]),
      pltpu.make_async_copy(
          x_any.at[b2, pl.ds(bot, 1)],
          buf.at[slot, pl.ds(R + 1, 1), pl.ds(1, W)], sem.at[slot]),
  )


def _fetch_tile(x_any, buf, sem, *, R, W, H):
  bi = pl.program_id(0)
  t = pl.program_id(1)
  nt = pl.num_programs(1)
  total = pl.num_programs(0) * nt
  step = bi * nt + t
  slot = lax.rem(step, 2)
  C = buf.shape[-1]

  @pl.when(step == 0)
  def _():
    for c in _halo_copies(x_any, buf, sem, step, nt, R=R, W=W, H=H):
      c.start()

  @pl.when(step + 1 < total)
  def _():
    for c in _halo_copies(x_any, buf, sem, step + 1, nt, R=R, W=W, H=H):
      c.start()

  for c in _halo_copies(x_any, buf, sem, step, nt, R=R, W=W, H=H):
    c.wait()

  @pl.when(step <= 1)
  def _():
    buf[pl.ds(slot, 1), :, 0:1, :] = jnp.zeros((1, R + 2, 1, C), jnp.float32)
    buf[pl.ds(slot, 1), :, W + 1:W + 2, :] = jnp.zeros(
        (1, R + 2, 1, C), jnp.float32)

  @pl.when(t == 0)
  def _():
    buf[pl.ds(slot, 1), 0:1, :, :] = jnp.zeros((1, 1, W + 2, C), jnp.float32)

  @pl.when(t == nt - 1)
  def _():
    buf[pl.ds(slot, 1), R + 1:R + 2, :, :] = jnp.zeros(
        (1, 1, W + 2, C), jnp.float32)

  return buf[slot]


def _fetch_tile_nchw(x_any, bufs, sem, *, R, W, H):
  buf_m, buf_t, buf_b = bufs
  bi = pl.program_id(0)
  t = pl.program_id(1)
  nt = pl.num_programs(1)
  total = pl.num_programs(0) * nt
  step = bi * nt + t
  slot = lax.rem(step, 2)

  def copies(s):
    b2 = s // nt
    t2 = lax.rem(s, nt)
    sl = lax.rem(s, 2)
    row0 = t2 * R
    top = jnp.maximum(row0 - 1, 0)
    bot = jnp.minimum(row0 + R, H - 1)
    return (
        pltpu.make_async_copy(
            x_any.at[b2, :, pl.ds(row0, R)], buf_m.at[sl], sem.at[sl]),
        pltpu.make_async_copy(
            x_any.at[b2, :, pl.ds(top, 1)], buf_t.at[sl], sem.at[sl]),
        pltpu.make_async_copy(
            x_any.at[b2, :, pl.ds(bot, 1)], buf_b.at[sl], sem.at[sl]),
    )

  @pl.when(step == 0)
  def _():
    for c in copies(step):
      c.start()

  @pl.when(step + 1 < total)
  def _():
    for c in copies(step + 1):
      c.start()

  for c in copies(step):
    c.wait()

  mt = jnp.where(t == 0, 0.0, 1.0).astype(jnp.float32)
  mb = jnp.where(t == nt - 1, 0.0, 1.0).astype(jnp.float32)
  rows = [jnp.transpose(buf_t[slot, :, 0, :]) * mt]
  for r in range(R):
    rows.append(jnp.transpose(buf_m[slot, :, r, :]))
  rows.append(jnp.transpose(buf_b[slot, :, 0, :]) * mb)
  x = jnp.stack(rows, axis=0)
  return jnp.pad(x, ((0, 0), (1, 1), (0, 0)))


_TAPS = [(dy, dx) for dy in range(3) for dx in range(3)]


def _tap_group_size(C):
  return 2 if C == 64 else max(1, 256 // C)


def _tap_gemm(x, wg_ref, wr_ref, *, R, W, C, Cout):
  g = _tap_group_size(C)
  nf = 9 // g
  acc = jnp.zeros((R * W, Cout), jnp.float32)
  for p in range(nf):
    parts = []
    for j in range(g):
      dy, dx = _TAPS[p * g + j]
      parts.append(x[dy:dy + R, dx:dx + W, :].reshape(R * W, C))
    xs = jnp.concatenate(parts, axis=1) if g > 1 else parts[0]
    acc = acc + jnp.dot(xs, wg_ref[p], preferred_element_type=jnp.float32)
  for q in range(9 - nf * g):
    dy, dx = _TAPS[nf * g + q]
    xs = x[dy:dy + R, dx:dx + W, :].reshape(R * W, C)
    acc = acc + jnp.dot(xs, wr_ref[q], preferred_element_type=jnp.float32)
  return acc


def _split_taps(w9):
  _, C, Cout = w9.shape
  g = _tap_group_size(C)
  nf = 9 // g
  wg = w9[:nf * g].reshape(nf, g * C, Cout)
  wr = w9[nf * g:]
  return wg, wr


def _conv_body(wg_ref, wr_ref, b_ref, x_any, o_ref, *scratch, R, C, Cout, W,
               H, nchw_in=False):
  if nchw_in:
    buf_m, buf_t, buf_b, sem = scratch
    x = _fetch_tile_nchw(x_any, (buf_m, buf_t, buf_b), sem, R=R, W=W, H=H)
  else:
    buf, sem = scratch
    x = _fetch_tile(x_any, buf, sem, R=R, W=W, H=H)
  acc = _tap_gemm(x, wg_ref, wr_ref, R=R, W=W, C=C, Cout=Cout)
  y = jnp.maximum(acc + b_ref[0], 0.0)
  o_ref[0] = y.reshape(R, W, Cout)


def _conv3x3_relu(x, w9, bias, *, R, nchw_in=False):
  if nchw_in:
    B, C, H, W = x.shape
  else:
    B, H, W, C = x.shape
  Cout = w9.shape[-1]
  T = H // R
  body = functools.partial(_conv_body, R=R, C=C, Cout=Cout, W=W, H=H,
                           nchw_in=nchw_in)
  if nchw_in:
    bufs = [
        pltpu.VMEM((2, C, R, W), jnp.float32),
        pltpu.VMEM((2, C, 1, W), jnp.float32),
        pltpu.VMEM((2, C, 1, W), jnp.float32),
    ]
  else:
    bufs = [pltpu.VMEM((2, R + 2, W + 2, C), jnp.float32)]
  wg, wr = _split_taps(w9)
  return pl.pallas_call(
      body,
      grid=(B, T),
      in_specs=[
          pl.BlockSpec(wg.shape, lambda b, t: (0, 0, 0)),
          pl.BlockSpec(wr.shape, lambda b, t: (0, 0, 0)),
          pl.BlockSpec((1, Cout), lambda b, t: (0, 0)),
          pl.BlockSpec(memory_space=pl.ANY),
      ],
      out_specs=pl.BlockSpec((1, R, W, Cout), lambda b, t: (b, t, 0, 0)),
      out_shape=jax.ShapeDtypeStruct((B, H, W, Cout), jnp.float32),
      scratch_shapes=bufs + [pltpu.SemaphoreType.DMA((2,))],
      compiler_params=pltpu.CompilerParams(
          dimension_semantics=("arbitrary", "arbitrary")),
  )(wg, wr, bias, x)


def _stage3_body(wg9_ref, wr9_ref, b_ref, wn_ref, bn_ref, wg_ref, bg_ref,
                 wf_ref, bf_ref, noise_ref, x_any, logits_ref, buf, sem,
                 acc_ref, *, R, C, Cout, W, H):
  bi = pl.program_id(0)
  t = pl.program_id(1)
  nt = pl.num_programs(1)
  x = _fetch_tile(x_any, buf, sem, R=R, W=W, H=H)
  acc = _tap_gemm(x, wg9_ref, wr9_ref, R=R, W=W, C=C, Cout=Cout)
  h = jnp.maximum(acc + b_ref[0], 0.0)
  nm = jnp.dot(h, wn_ref[...], preferred_element_type=jnp.float32)
  nm = nm + bn_ref[0]
  contrib = h * (1.0 + nm)
  psum = jnp.sum(contrib, axis=0)[None, :]

  @pl.when(t == 0)
  def _():
    acc_ref[...] = psum

  @pl.when(t > 0)
  def _():
    acc_ref[...] = acc_ref[...] + psum

  @pl.when(t == nt - 1)
  def _():
    n_pix = jnp.float32(224 * 224)
    p = acc_ref[...] / n_pix
    clean = jnp.dot(p, wg_ref[...],
                    preferred_element_type=jnp.float32) + bg_ref[...]
    raw = jnp.dot(p, wf_ref[...],
                  preferred_element_type=jnp.float32) + bf_ref[...]
    softplus = jnp.maximum(raw, 0.0) + jnp.log1p(jnp.exp(-jnp.abs(raw)))
    std = softplus + jnp.float32(0.01)
    logits_ref[pl.ds(bi, 1), :] = clean + noise_ref[pl.ds(bi, 1), :] * std


def _stage3(x, w9, bias, wn, bn, wg_t, bg, wf_t, bf, noise, *, R):
  B, H, W, C = x.shape
  Cout = w9.shape[-1]
  E = wg_t.shape[-1]
  T = H // R
  body = functools.partial(_stage3_body, R=R, C=C, Cout=Cout, W=W, H=H)
  wg9, wr9 = _split_taps(w9)
  return pl.pallas_call(
      body,
      grid=(B, T),
      in_specs=[
          pl.BlockSpec(wg9.shape, lambda b, t: (0, 0, 0)),
          pl.BlockSpec(wr9.shape, lambda b, t: (0, 0, 0)),
          pl.BlockSpec((1, Cout), lambda b, t: (0, 0)),
          pl.BlockSpec((Cout, 1), lambda b, t: (0, 0)),
          pl.BlockSpec((1, 1), lambda b, t: (0, 0)),
          pl.BlockSpec((Cout, E), lambda b, t: (0, 0)),
          pl.BlockSpec((1, E), lambda b, t: (0, 0)),
          pl.BlockSpec((Cout, E), lambda b, t: (0, 0)),
          pl.BlockSpec((1, E), lambda b, t: (0, 0)),
          pl.BlockSpec((B, E), lambda b, t: (0, 0)),
          pl.BlockSpec(memory_space=pl.ANY),
      ],
      out_specs=pl.BlockSpec((B, E), lambda b, t: (0, 0)),
      out_shape=jax.ShapeDtypeStruct((B, E), jnp.float32),
      scratch_shapes=[
          pltpu.VMEM((2, R + 2, W + 2, C), jnp.float32),
          pltpu.SemaphoreType.DMA((2,)),
          pltpu.VMEM((1, Cout), jnp.float32),
      ],
      compiler_params=pltpu.CompilerParams(
          dimension_semantics=("arbitrary", "arbitrary")),
  )(wg9, wr9, bias, wn, bn, wg_t, bg, wf_t, bf, noise, x)


def _make_router(B, E):
  mesh = plsc.VectorSubcoreMesh(core_axis_name="c", subcore_axis_name="s")

  @functools.partial(
      pl.kernel,
      mesh=mesh,
      out_type=(
          jax.ShapeDtypeStruct((B, E), jnp.float32),
          jax.ShapeDtypeStruct((E,), jnp.float32),
      ),
      scratch_types=[
          pltpu.VMEM((B, E), jnp.float32),
          pltpu.VMEM((B, E), jnp.float32),
          pltpu.VMEM((E,), jnp.float32),
      ],
  )
  def router(logits_hbm, gates_hbm, load_hbm, logits_v, gates_v, load_v):
    cid = lax.axis_index("c")
    sid = lax.axis_index("s")

    @pl.when(jnp.logical_and(cid == 0, sid == 0))
    def _():
      pltpu.sync_copy(logits_hbm, logits_v)
      idx = lax.iota(jnp.int32, E)
      load_acc = jnp.zeros((E,), jnp.float32)
      neg_inf = jnp.float32(-jnp.inf)
      for b in range(B):
        v = logits_v[b]
        m1 = neg_inf
        i1 = jnp.int32(-1)
        for e in range(E):
          le = v[e]
          better = le > m1
          m1 = jnp.where(better, le, m1)
          i1 = jnp.where(better, jnp.int32(e), i1)
        m2 = neg_inf
        i2 = jnp.int32(-1)
        for e in range(E):
          le = v[e]
          better = jnp.logical_and(le > m2, jnp.int32(e) != i1)
          m2 = jnp.where(better, le, m2)
          i2 = jnp.where(better, jnp.int32(e), i2)
        e2 = jnp.exp(jnp.broadcast_to(m2 - m1, (E,)))
        denom = 1.0 + e2
        g1 = 1.0 / denom
        g2 = e2 / denom
        row = jnp.where(idx == i1, g1,
                        jnp.where(idx == i2, g2, jnp.float32(0.0)))
        gates_v[b] = row
        load_acc = load_acc + row
      load_v[...] = load_acc
      pltpu.sync_copy(gates_v, gates_hbm)
      pltpu.sync_copy(load_v, load_hbm)

  return router


def _taps(w):
  o, i, _, _ = w.shape
  return w.transpose(2, 3, 1, 0).reshape(9, i, o)


def kernel(x, W1, b1, W2, b2, W3, b3, Wn, bn, Wg, bg, Wf, bf):
  B = x.shape[0]
  E = Wg.shape[0]

  h1 = _conv3x3_relu(x, _taps(W1), b1[None, :], R=32, nchw_in=True)
  h2 = _conv3x3_relu(h1, _taps(W2), b2[None, :], R=32)
  noise = jax.random.normal(jax.random.key(42), (B, E), dtype=jnp.float32)
  logits = _stage3(
      h2, _taps(W3), b3[None, :],
      Wn[:, :, 0, 0].T,
      bn[None, :],
      Wg.T, bg[None, :], Wf.T, bf[None, :], noise, R=32)
  gates, load = _make_router(B, E)(logits)
  return gates, load


# speedup vs baseline: 1.0726x; 1.0726x over previous
import functools

import jax
import jax.numpy as jnp
from jax import lax
from jax.experimental import pallas as pl
from jax.experimental.pallas import tpu as pltpu
from jax.experimental.pallas import tpu_sc as plsc


def _halo_copies(x_any, buf, sem, s, nt, *, R, W, H):
  b2 = s // nt
  t2 = lax.rem(s, nt)
  slot = lax.rem(s, 2)
  row0 = t2 * R
  top = jnp.maximum(row0 - 1, 0)
  bot = jnp.minimum(row0 + R, H - 1)
  return (
      pltpu.make_async_copy(
          x_any.at[b2, pl.ds(row0, R)],
          buf.at[slot, pl.ds(1, R), pl.ds(1, W)], sem.at[slot]),
      pltpu.make_async_copy(
          x_any.at[b2, pl.ds(top, 1)],
          buf.at[slot, pl.ds(0, 1), pl.ds(1, W)], sem.at[slot]),
      pltpu.make_async_copy(
          x_any.at[b2, pl.ds(bot, 1)],
          buf.at[slot, pl.ds(R + 1, 1), pl.ds(1, W)], sem.at[slot]),
  )


def _fetch_tile(x_any, buf, sem, *, R, W, H):
  bi = pl.program_id(0)
  t = pl.program_id(1)
  nt = pl.num_programs(1)
  total = pl.num_programs(0) * nt
  step = bi * nt + t
  slot = lax.rem(step, 2)
  C = buf.shape[-1]

  @pl.when(step == 0)
  def _():
    for c in _halo_copies(x_any, buf, sem, step, nt, R=R, W=W, H=H):
      c.start()

  @pl.when(step + 1 < total)
  def _():
    for c in _halo_copies(x_any, buf, sem, step + 1, nt, R=R, W=W, H=H):
      c.start()

  for c in _halo_copies(x_any, buf, sem, step, nt, R=R, W=W, H=H):
    c.wait()

  @pl.when(step <= 1)
  def _():
    buf[pl.ds(slot, 1), :, 0:1, :] = jnp.zeros((1, R + 2, 1, C), jnp.float32)
    buf[pl.ds(slot, 1), :, W + 1:W + 2, :] = jnp.zeros(
        (1, R + 2, 1, C), jnp.float32)

  @pl.when(t == 0)
  def _():
    buf[pl.ds(slot, 1), 0:1, :, :] = jnp.zeros((1, 1, W + 2, C), jnp.float32)

  @pl.when(t == nt - 1)
  def _():
    buf[pl.ds(slot, 1), R + 1:R + 2, :, :] = jnp.zeros(
        (1, 1, W + 2, C), jnp.float32)

  return buf[slot]


def _fetch_tile_nchw(x_any, bufs, sem, *, R, W, H):
  buf_m, buf_t, buf_b = bufs
  bi = pl.program_id(0)
  t = pl.program_id(1)
  nt = pl.num_programs(1)
  total = pl.num_programs(0) * nt
  step = bi * nt + t
  slot = lax.rem(step, 2)

  def copies(s):
    b2 = s // nt
    t2 = lax.rem(s, nt)
    sl = lax.rem(s, 2)
    row0 = t2 * R
    top = jnp.maximum(row0 - 1, 0)
    bot = jnp.minimum(row0 + R, H - 1)
    return (
        pltpu.make_async_copy(
            x_any.at[b2, :, pl.ds(row0, R)], buf_m.at[sl], sem.at[sl]),
        pltpu.make_async_copy(
            x_any.at[b2, :, pl.ds(top, 1)], buf_t.at[sl], sem.at[sl]),
        pltpu.make_async_copy(
            x_any.at[b2, :, pl.ds(bot, 1)], buf_b.at[sl], sem.at[sl]),
    )

  @pl.when(step == 0)
  def _():
    for c in copies(step):
      c.start()

  @pl.when(step + 1 < total)
  def _():
    for c in copies(step + 1):
      c.start()

  for c in copies(step):
    c.wait()

  mt = jnp.where(t == 0, 0.0, 1.0).astype(jnp.float32)
  mb = jnp.where(t == nt - 1, 0.0, 1.0).astype(jnp.float32)
  rows = [jnp.transpose(buf_t[slot, :, 0, :]) * mt]
  for r in range(R):
    rows.append(jnp.transpose(buf_m[slot, :, r, :]))
  rows.append(jnp.transpose(buf_b[slot, :, 0, :]) * mb)
  x = jnp.stack(rows, axis=0)
  return jnp.pad(x, ((0, 0), (1, 1), (0, 0)))


_TAPS = [(dy, dx) for dy in range(3) for dx in range(3)]


def _tap_group_size(C):
  return max(1, 256 // C)


def _tap_gemm(x, wg_ref, wr_ref, *, R, W, C, Cout):
  g = _tap_group_size(C)
  nf = 9 // g
  acc = jnp.zeros((R * W, Cout), jnp.float32)
  for p in range(nf):
    parts = []
    for j in range(g):
      dy, dx = _TAPS[p * g + j]
      parts.append(x[dy:dy + R, dx:dx + W, :].reshape(R * W, C))
    xs = jnp.concatenate(parts, axis=1) if g > 1 else parts[0]
    acc = acc + jnp.dot(xs, wg_ref[p], preferred_element_type=jnp.float32)
  for q in range(9 - nf * g):
    dy, dx = _TAPS[nf * g + q]
    xs = x[dy:dy + R, dx:dx + W, :].reshape(R * W, C)
    acc = acc + jnp.dot(xs, wr_ref[q], preferred_element_type=jnp.float32)
  return acc


def _split_taps(w9):
  _, C, Cout = w9.shape
  g = _tap_group_size(C)
  nf = 9 // g
  wg = w9[:nf * g].reshape(nf, g * C, Cout)
  wr = w9[nf * g:]
  return wg, wr


def _conv_body(wg_ref, wr_ref, b_ref, x_any, o_ref, *scratch, R, C, Cout, W,
               H, nchw_in=False):
  if nchw_in:
    buf_m, buf_t, buf_b, sem = scratch
    x = _fetch_tile_nchw(x_any, (buf_m, buf_t, buf_b), sem, R=R, W=W, H=H)
  else:
    buf, sem = scratch
    x = _fetch_tile(x_any, buf, sem, R=R, W=W, H=H)
  acc = _tap_gemm(x, wg_ref, wr_ref, R=R, W=W, C=C, Cout=Cout)
  y = jnp.maximum(acc + b_ref[0], 0.0)
  o_ref[0] = y.reshape(R, W, Cout)


def _conv3x3_relu(x, w9, bias, *, R, nchw_in=False):
  if nchw_in:
    B, C, H, W = x.shape
  else:
    B, H, W, C = x.shape
  Cout = w9.shape[-1]
  T = H // R
  body = functools.partial(_conv_body, R=R, C=C, Cout=Cout, W=W, H=H,
                           nchw_in=nchw_in)
  if nchw_in:
    bufs = [
        pltpu.VMEM((2, C, R, W), jnp.float32),
        pltpu.VMEM((2, C, 1, W), jnp.float32),
        pltpu.VMEM((2, C, 1, W), jnp.float32),
    ]
  else:
    bufs = [pltpu.VMEM((2, R + 2, W + 2, C), jnp.float32)]
  wg, wr = _split_taps(w9)
  return pl.pallas_call(
      body,
      grid=(B, T),
      in_specs=[
          pl.BlockSpec(wg.shape, lambda b, t: (0, 0, 0)),
          pl.BlockSpec(wr.shape, lambda b, t: (0, 0, 0)),
          pl.BlockSpec((1, Cout), lambda b, t: (0, 0)),
          pl.BlockSpec(memory_space=pl.ANY),
      ],
      out_specs=pl.BlockSpec((1, R, W, Cout), lambda b, t: (b, t, 0, 0)),
      out_shape=jax.ShapeDtypeStruct((B, H, W, Cout), jnp.float32),
      scratch_shapes=bufs + [pltpu.SemaphoreType.DMA((2,))],
      compiler_params=pltpu.CompilerParams(
          dimension_semantics=("arbitrary", "arbitrary")),
  )(wg, wr, bias, x)


def _stage3_body(wg9_ref, wr9_ref, b_ref, wn_ref, bn_ref, wg_ref, bg_ref,
                 wf_ref, bf_ref, noise_ref, x_any, logits_ref, buf, sem,
                 acc_ref, *, R, C, Cout, W, H):
  bi = pl.program_id(0)
  t = pl.program_id(1)
  nt = pl.num_programs(1)
  x = _fetch_tile(x_any, buf, sem, R=R, W=W, H=H)
  acc = _tap_gemm(x, wg9_ref, wr9_ref, R=R, W=W, C=C, Cout=Cout)
  h = jnp.maximum(acc + b_ref[0], 0.0)
  nm = jnp.dot(h, wn_ref[...], preferred_element_type=jnp.float32)
  nm = nm + bn_ref[0]
  contrib = h * (1.0 + nm)
  psum = jnp.sum(contrib, axis=0)[None, :]

  @pl.when(t == 0)
  def _():
    acc_ref[...] = psum

  @pl.when(t > 0)
  def _():
    acc_ref[...] = acc_ref[...] + psum

  @pl.when(t == nt - 1)
  def _():
    n_pix = jnp.float32(224 * 224)
    p = acc_ref[...] / n_pix
    clean = jnp.dot(p, wg_ref[...],
                    preferred_element_type=jnp.float32) + bg_ref[...]
    raw = jnp.dot(p, wf_ref[...],
                  preferred_element_type=jnp.float32) + bf_ref[...]
    softplus = jnp.maximum(raw, 0.0) + jnp.log1p(jnp.exp(-jnp.abs(raw)))
    std = softplus + jnp.float32(0.01)
    logits_ref[pl.ds(bi, 1), :] = clean + noise_ref[pl.ds(bi, 1), :] * std


def _stage3(x, w9, bias, wn, bn, wg_t, bg, wf_t, bf, noise, *, R):
  B, H, W, C = x.shape
  Cout = w9.shape[-1]
  E = wg_t.shape[-1]
  T = H // R
  body = functools.partial(_stage3_body, R=R, C=C, Cout=Cout, W=W, H=H)
  wg9, wr9 = _split_taps(w9)
  return pl.pallas_call(
      body,
      grid=(B, T),
      in_specs=[
          pl.BlockSpec(wg9.shape, lambda b, t: (0, 0, 0)),
          pl.BlockSpec(wr9.shape, lambda b, t: (0, 0, 0)),
          pl.BlockSpec((1, Cout), lambda b, t: (0, 0)),
          pl.BlockSpec((Cout, 1), lambda b, t: (0, 0)),
          pl.BlockSpec((1, 1), lambda b, t: (0, 0)),
          pl.BlockSpec((Cout, E), lambda b, t: (0, 0)),
          pl.BlockSpec((1, E), lambda b, t: (0, 0)),
          pl.BlockSpec((Cout, E), lambda b, t: (0, 0)),
          pl.BlockSpec((1, E), lambda b, t: (0, 0)),
          pl.BlockSpec((B, E), lambda b, t: (0, 0)),
          pl.BlockSpec(memory_space=pl.ANY),
      ],
      out_specs=pl.BlockSpec((B, E), lambda b, t: (0, 0)),
      out_shape=jax.ShapeDtypeStruct((B, E), jnp.float32),
      scratch_shapes=[
          pltpu.VMEM((2, R + 2, W + 2, C), jnp.float32),
          pltpu.SemaphoreType.DMA((2,)),
          pltpu.VMEM((1, Cout), jnp.float32),
      ],
      compiler_params=pltpu.CompilerParams(
          dimension_semantics=("arbitrary", "arbitrary")),
  )(wg9, wr9, bias, wn, bn, wg_t, bg, wf_t, bf, noise, x)


def _make_router(B, E):
  mesh = plsc.VectorSubcoreMesh(core_axis_name="c", subcore_axis_name="s")

  @functools.partial(
      pl.kernel,
      mesh=mesh,
      out_type=(
          jax.ShapeDtypeStruct((B, E), jnp.float32),
          jax.ShapeDtypeStruct((E,), jnp.float32),
      ),
      scratch_types=[
          pltpu.VMEM((B, E), jnp.float32),
          pltpu.VMEM((B, E), jnp.float32),
          pltpu.VMEM((E,), jnp.float32),
      ],
  )
  def router(logits_hbm, gates_hbm, load_hbm, logits_v, gates_v, load_v):
    cid = lax.axis_index("c")
    sid = lax.axis_index("s")

    @pl.when(jnp.logical_and(cid == 0, sid == 0))
    def _():
      pltpu.sync_copy(logits_hbm, logits_v)
      idx = lax.iota(jnp.int32, E)
      load_acc = jnp.zeros((E,), jnp.float32)
      neg_inf = jnp.float32(-jnp.inf)
      for b in range(B):
        v = logits_v[b]
        m1 = neg_inf
        i1 = jnp.int32(-1)
        for e in range(E):
          le = v[e]
          better = le > m1
          m1 = jnp.where(better, le, m1)
          i1 = jnp.where(better, jnp.int32(e), i1)
        m2 = neg_inf
        i2 = jnp.int32(-1)
        for e in range(E):
          le = v[e]
          better = jnp.logical_and(le > m2, jnp.int32(e) != i1)
          m2 = jnp.where(better, le, m2)
          i2 = jnp.where(better, jnp.int32(e), i2)
        e2 = jnp.exp(jnp.broadcast_to(m2 - m1, (E,)))
        denom = 1.0 + e2
        g1 = 1.0 / denom
        g2 = e2 / denom
        row = jnp.where(idx == i1, g1,
                        jnp.where(idx == i2, g2, jnp.float32(0.0)))
        gates_v[b] = row
        load_acc = load_acc + row
      load_v[...] = load_acc
      pltpu.sync_copy(gates_v, gates_hbm)
      pltpu.sync_copy(load_v, load_hbm)

  return router


def _taps(w):
  o, i, _, _ = w.shape
  return w.transpose(2, 3, 1, 0).reshape(9, i, o)


def kernel(x, W1, b1, W2, b2, W3, b3, Wn, bn, Wg, bg, Wf, bf):
  B = x.shape[0]
  E = Wg.shape[0]

  h1 = _conv3x3_relu(x, _taps(W1), b1[None, :], R=56, nchw_in=True)
  h2 = _conv3x3_relu(h1, _taps(W2), b2[None, :], R=56)
  noise = jax.random.normal(jax.random.key(42), (B, E), dtype=jnp.float32)
  logits = _stage3(
      h2, _taps(W3), b3[None, :],
      Wn[:, :, 0, 0].T,
      bn[None, :],
      Wg.T, bg[None, :], Wf.T, bf[None, :], noise, R=56)
  gates, load = _make_router(B, E)(logits)
  return gates, load
